# Initial kernel scaffold; baseline (speedup 1.0000x reference)
import jax
import jax.numpy as jnp
from jax.experimental import pallas as pl


def kernel(nodes, edges, params, senders, receivers, n_nodes):
    raise NotImplementedError("write your pallas kernel here")


# speedup vs baseline: 1.6682x; 1.6682x over previous
import functools

import jax
import jax.numpy as jnp
from jax import lax
from jax.experimental import pallas as pl
from jax.experimental.pallas import tpu as pltpu
import jax.experimental.pallas.tpu_sc as plsc

_F32 = jnp.float32
_NC, _NS = 2, 16
_NW = _NC * _NS
_CHUNK = 128
_STEPS = 8


def _pc(body, grid, in_specs, out_specs, out_shape):
    return pl.pallas_call(
        body, grid=grid, in_specs=in_specs, out_specs=out_specs,
        out_shape=out_shape)


def _full(shape):
    return pl.BlockSpec(shape, lambda i: (0,) * len(shape))


def _ln(h, g, be):
    mu = jnp.mean(h, axis=-1, keepdims=True)
    c = h - mu
    var = jnp.mean(c * c, axis=-1, keepdims=True)
    return c / jnp.sqrt(var + 1e-5) * g + be


def _enc_body(x_ref, w0, b0, w1, b1, w2, b2, g, be, o_ref, *, k1_matmul):
    x = x_ref[...]
    if k1_matmul:
        h = jnp.dot(x, w0[...], preferred_element_type=_F32, precision=lax.Precision.HIGHEST) + b0[...]
    else:
        h = x * w0[...] + b0[...]
    h = jnp.maximum(h, 0.0)
    h = jnp.maximum(jnp.dot(h, w1[...], preferred_element_type=_F32, precision=lax.Precision.HIGHEST) + b1[...], 0.0)
    h = jnp.dot(h, w2[...], preferred_element_type=_F32, precision=lax.Precision.HIGHEST) + b2[...]
    o_ref[...] = _ln(h, g[...], be[...])


def _encoder(x, p, block_rows):
    (w0, b0), (w1, b1), (w2, b2) = p["layers"]
    g, be = p["ln"]
    rows, fi = x.shape
    fo = w2.shape[1]
    k1 = fi > 1
    body = functools.partial(_enc_body, k1_matmul=k1)
    return _pc(
        body, (rows // block_rows,),
        [pl.BlockSpec((block_rows, fi), lambda i: (i, 0)),
         _full(w0.shape), _full((1, b0.shape[0])),
         _full(w1.shape), _full((1, b1.shape[0])),
         _full(w2.shape), _full((1, b2.shape[0])),
         _full((1, fo)), _full((1, fo))],
        pl.BlockSpec((block_rows, fo), lambda i: (i, 0)),
        jax.ShapeDtypeStruct((rows, fo), _F32),
    )(x, w0, b0[None], w1, b1[None], w2, b2[None], g[None], be[None])


def _edge_body(e_ref, gs_ref, gr_ref, w1e, w1s, w1r, b1, w2, b2, w3, b3,
               g, be, o_ref):
    x = jnp.concatenate([e_ref[...], gs_ref[...], gr_ref[...]], axis=1)
    w1 = jnp.concatenate([w1e[...], w1s[...], w1r[...]], axis=0)
    h = jnp.dot(x, w1, preferred_element_type=_F32, precision=lax.Precision.HIGHEST) + b1[...]
    h = jnp.maximum(h, 0.0)
    h = jnp.maximum(jnp.dot(h, w2[...], preferred_element_type=_F32, precision=lax.Precision.HIGHEST) + b2[...], 0.0)
    h = jnp.dot(h, w3[...], preferred_element_type=_F32, precision=lax.Precision.HIGHEST) + b3[...]
    o_ref[...] = _ln(h, g[...], be[...]) * 2.0


def _edge_mlp(e, gs, gr, wsplit, block_rows):
    w1e, w1s, w1r, b1, w2, b2, w3, b3, g, be = wsplit
    rows = e.shape[0]
    spec = pl.BlockSpec((block_rows, 128), lambda i: (i, 0))
    return _pc(
        _edge_body, (rows // block_rows,),
        [spec, spec, spec,
         _full((128, 128)), _full((128, 128)), _full((128, 128)), _full((1, 128)),
         _full((128, 128)), _full((1, 128)),
         _full((128, 128)), _full((1, 128)),
         _full((1, 128)), _full((1, 128))],
        spec,
        jax.ShapeDtypeStruct((rows, 128), _F32),
    )(e, gs, gr, w1e, w1s, w1r, b1, w2, b2, w3, b3, g, be)


def _node_body(n_ref, s0_ref, s1_ref, w1n, w1s, b1, w2, b2, w3, b3, g, be, o_ref):
    s = (s0_ref[...] + s1_ref[...]) * 0.5
    x = jnp.concatenate([n_ref[...], s], axis=1)
    w1 = jnp.concatenate([w1n[...], w1s[...]], axis=0)
    h = jnp.dot(x, w1, preferred_element_type=_F32, precision=lax.Precision.HIGHEST) + b1[...]
    h = jnp.maximum(h, 0.0)
    h = jnp.maximum(jnp.dot(h, w2[...], preferred_element_type=_F32, precision=lax.Precision.HIGHEST) + b2[...], 0.0)
    h = jnp.dot(h, w3[...], preferred_element_type=_F32, precision=lax.Precision.HIGHEST) + b3[...]
    o_ref[...] = _ln(h, g[...], be[...]) * 2.0


def _node_mlp(n, s0, s1, wsplit, block_rows):
    w1n, w1s, b1, w2, b2, w3, b3, g, be = wsplit
    rows = n.shape[0]
    spec = pl.BlockSpec((block_rows, 128), lambda i: (i, 0))
    return _pc(
        _node_body, (rows // block_rows,),
        [spec, spec, spec,
         _full((128, 128)), _full((128, 128)), _full((1, 128)),
         _full((128, 128)), _full((1, 128)),
         _full((128, 128)), _full((1, 128)),
         _full((1, 128)), _full((1, 128))],
        spec,
        jax.ShapeDtypeStruct((rows, 128), _F32),
    )(n, s0, s1, w1n, w1s, b1, w2, b2, w3, b3, g, be)


def _dec_body(x_ref, w0, b0, w1, b1, w2, b2, o_ref):
    h = jnp.maximum(jnp.dot(x_ref[...], w0[...], preferred_element_type=_F32, precision=lax.Precision.HIGHEST) + b0[...], 0.0)
    h = jnp.maximum(jnp.dot(h, w1[...], preferred_element_type=_F32, precision=lax.Precision.HIGHEST) + b1[...], 0.0)
    o_ref[...] = jnp.dot(h, w2[...], preferred_element_type=_F32, precision=lax.Precision.HIGHEST) + b2[...]


def _decoder(x, p, block_rows):
    (w0, b0), (w1, b1), (w2, b2) = p["layers"]
    rows = x.shape[0]
    fo = w2.shape[1]
    w2p = jnp.zeros((w2.shape[0], 128), _F32).at[:, :fo].set(w2)
    b2p = jnp.zeros((128,), _F32).at[:fo].set(b2)
    return _pc(
        _dec_body, (rows // block_rows,),
        [pl.BlockSpec((block_rows, 128), lambda i: (i, 0)),
         _full(w0.shape), _full((1, b0.shape[0])),
         _full(w1.shape), _full((1, b1.shape[0])),
         _full((w2.shape[0], 128)), _full((1, 128))],
        pl.BlockSpec((block_rows, 128), lambda i: (i, 0)),
        jax.ShapeDtypeStruct((rows, 128), _F32),
    )(x, w0, b0[None], w1, b1[None], w2p, b2p[None])[:, :fo]


def _sc_gather(table, s_idx, r_idx):
    e = s_idx.shape[0]
    nch = e // _CHUNK
    rem = nch % _NW
    mesh = plsc.VectorSubcoreMesh(core_axis_name="c", subcore_axis_name="s")

    @functools.partial(
        pl.kernel,
        out_type=(jax.ShapeDtypeStruct((e, 128), _F32),
                  jax.ShapeDtypeStruct((e, 128), _F32)),
        mesh=mesh,
        scratch_types=[
            pltpu.VMEM((_CHUNK,), jnp.int32),
            pltpu.VMEM((_CHUNK,), jnp.int32),
            pltpu.VMEM((_CHUNK, 128), _F32),
            pltpu.VMEM((_CHUNK, 128), _F32),
            pltpu.SemaphoreType.DMA,
            pltpu.SemaphoreType.DMA,
        ],
    )
    def k(table_hbm, sidx_hbm, ridx_hbm, gs_hbm, gr_hbm,
          idxs_v, idxr_v, rs_v, rr_v, sem_s, sem_r):
        w = lax.axis_index("s") * _NC + lax.axis_index("c")
        n_mine = jnp.where(w < rem, nch // _NW + 1, nch // _NW)

        def body(i, carry):
            base = (w + i * _NW) * _CHUNK
            pltpu.sync_copy(sidx_hbm.at[pl.ds(base, _CHUNK)], idxs_v)
            pltpu.sync_copy(ridx_hbm.at[pl.ds(base, _CHUNK)], idxr_v)
            cs = pltpu.async_copy(table_hbm.at[idxs_v], rs_v, sem_s)
            cr = pltpu.async_copy(table_hbm.at[idxr_v], rr_v, sem_r)
            cs.wait()
            cr.wait()
            pltpu.sync_copy(rs_v, gs_hbm.at[pl.ds(base, _CHUNK)])
            pltpu.sync_copy(rr_v, gr_hbm.at[pl.ds(base, _CHUNK)])
            return carry

        lax.fori_loop(0, n_mine, body, 0)

    return k(table, s_idx, r_idx)


def _sc_scatter_add(vals, r_idx, zeros_pad):
    e = vals.shape[0]
    npad = zeros_pad.shape[0]
    rows_t = npad // _NS
    nch = e // _CHUNK
    rem = nch % _NW
    mesh = plsc.VectorSubcoreMesh(core_axis_name="c", subcore_axis_name="s")

    @functools.partial(
        pl.kernel,
        out_type=(jax.ShapeDtypeStruct((npad, 128), _F32),
                  jax.ShapeDtypeStruct((npad, 128), _F32)),
        mesh=mesh,
        scratch_types=[
            pltpu.VMEM((_CHUNK,), jnp.int32),
            pltpu.VMEM((_CHUNK, 128), _F32),
            pltpu.VMEM_SHARED((npad, 128), _F32),
        ],
    )
    def k(vals_hbm, ridx_hbm, z_hbm, o0, o1, idx_v, rows_v, acc):
        c = lax.axis_index("c")
        s = lax.axis_index("s")
        w = s * _NC + c
        n_mine = jnp.where(w < rem, nch // _NW + 1, nch // _NW)
        sl = pl.ds(s * rows_t, rows_t)
        pltpu.sync_copy(z_hbm.at[sl], acc.at[sl])
        plsc.subcore_barrier()

        def body(i, carry):
            base = (w + i * _NW) * _CHUNK
            pltpu.sync_copy(ridx_hbm.at[pl.ds(base, _CHUNK)], idx_v)
            pltpu.sync_copy(vals_hbm.at[pl.ds(base, _CHUNK)], rows_v)
            pltpu.sync_copy(rows_v, acc.at[idx_v], add=True)
            return carry

        lax.fori_loop(0, n_mine, body, 0)
        plsc.subcore_barrier()

        @pl.when(c == 0)
        def _():
            pltpu.sync_copy(acc.at[sl], o0.at[sl])

        @pl.when(c == 1)
        def _():
            pltpu.sync_copy(acc.at[sl], o1.at[sl])

    return k(vals, r_idx, zeros_pad)


def kernel(nodes, edges, params, senders, receivers, n_nodes):
    del n_nodes
    n = nodes.shape[0]
    e = edges.shape[0]
    npad = ((n + _NS * 8 - 1) // (_NS * 8)) * (_NS * 8)

    bn = 1000 if n % 1000 == 0 else 8
    be = 2000 if e % 2000 == 0 else 8

    pe = params["edge_proc"]
    (w1, b1), (w2, b2), (w3, b3) = pe["layers"]
    ge, bee = pe["ln"]
    edge_w = (w1[:128], w1[128:256], w1[256:384], b1[None], w2, b2[None],
              w3, b3[None], ge[None], bee[None])

    pn = params["node_proc"]
    (nw1, nb1), (nw2, nb2), (nw3, nb3) = pn["layers"]
    gn, ben = pn["ln"]
    node_w = (nw1[:128], nw1[128:], nb1[None], nw2, nb2[None],
              nw3, nb3[None], gn[None], ben[None])

    h_nodes = _encoder(nodes, params["node_enc"], bn)
    h_edges = _encoder(edges, params["edge_enc"], be)

    zeros_pad = jnp.zeros((npad, 128), _F32)
    for _ in range(_STEPS):
        gs, gr = _sc_gather(h_nodes, senders, receivers)
        h_edges = _edge_mlp(h_edges, gs, gr, edge_w, be)
        s0, s1 = _sc_scatter_add(h_edges, receivers, zeros_pad)
        h_nodes = _node_mlp(h_nodes, s0[:n], s1[:n], node_w, bn)

    return _decoder(h_nodes, params["dec"], bn)
